# Initial kernel scaffold; baseline (speedup 1.0000x reference)
import jax
import jax.numpy as jnp
from jax.experimental import pallas as pl


def kernel(hidden_states, targets):
    raise NotImplementedError("write your pallas kernel here")


# speedup vs baseline: 1.5456x; 1.5456x over previous
import jax
import jax.numpy as jnp
from jax.experimental import pallas as pl
from jax.experimental.pallas import tpu as pltpu


def _tc_body(hs_ref, tg_ref, out_hs_ref, stats_ref):
    h = hs_ref[...]
    t = tg_ref[...]
    out_hs_ref[...] = h
    d = h - t
    stats_ref[0, :] = jnp.sum(d * d, axis=1)
    stats_ref[1, :] = jnp.sum(jnp.abs(h), axis=1)
    stats_ref[2, :] = jnp.sum(t, axis=1)


def _tc_stage(hs, tg, rows_per_block=1024):
    n, d = hs.shape
    grid = (n // rows_per_block,)
    out_hs, stats = pl.pallas_call(
        _tc_body,
        grid=grid,
        in_specs=[
            pl.BlockSpec((rows_per_block, d), lambda i: (i, 0)),
            pl.BlockSpec((rows_per_block, d), lambda i: (i, 0)),
        ],
        out_specs=[
            pl.BlockSpec((rows_per_block, d), lambda i: (i, 0)),
            pl.BlockSpec((3, rows_per_block), lambda i: (0, i)),
        ],
        out_shape=[
            jax.ShapeDtypeStruct((n, d), jnp.float32),
            jax.ShapeDtypeStruct((3, n), jnp.float32),
        ],
    )(hs, tg)
    return out_hs, stats


def kernel(hidden_states, targets):
    B, S, D = hidden_states.shape
    n = B * S
    hs = hidden_states.reshape(n, D)
    tg = targets.reshape(n, D)
    out_hs, stats = _tc_stage(hs, tg)
    sq_rows, ab_rows, tgsum = stats[0], stats[1], stats[2]
    mask = tgsum != 0
    n_elems = jnp.sum(mask).astype(jnp.float32) * D
    loss = jnp.sum(jnp.where(mask, sq_rows, 0.0)) / n_elems
    mab = jnp.sum(jnp.where(mask, ab_rows, 0.0)) / n_elems
    return (loss, mab, out_hs.reshape(B, S, D))
